# baseline (speedup 1.0000x reference)
import jax
import jax.numpy as jnp
from jax.experimental import pallas as pl

B, Q, T, C, K = 128, 900, 256, 91, 300


def _prob_body(logits_ref, pm_ref, prob_ref):
    x = logits_ref[0]
    sig = 1.0 / (1.0 + jnp.exp(-x))
    pm = pm_ref[...]
    prob = jax.lax.dot_general(
        sig, pm, (((1,), (1,)), ((), ())),
        preferred_element_type=jnp.float32)
    prob_ref[0] = prob


def _compute_prob(pred_logits, positive_map):
    return pl.pallas_call(
        _prob_body,
        grid=(B,),
        in_specs=[
            pl.BlockSpec((1, Q, T), lambda b: (b, 0, 0)),
            pl.BlockSpec((C, T), lambda b: (0, 0)),
        ],
        out_specs=pl.BlockSpec((1, Q, C), lambda b: (b, 0, 0)),
        out_shape=jax.ShapeDtypeStruct((B, Q, C), jnp.float32),
    )(pred_logits, positive_map)


def kernel(pred_logits, pred_boxes, target_sizes, positive_map):
    prob = _compute_prob(pred_logits, positive_map)
    topk_values, topk_indexes = jax.lax.top_k(prob.reshape(B, Q * C), K)
    scores = topk_values
    topk_boxes = topk_indexes // C
    labels = topk_indexes % C

    cx, cy, w, h = (pred_boxes[..., i] for i in range(4))
    boxes = jnp.stack([cx - 0.5 * w, cy - 0.5 * h, cx + 0.5 * w, cy + 0.5 * h],
                      axis=-1)
    idx = jnp.repeat(topk_boxes[:, :, None], 4, axis=2)
    boxes = jnp.take_along_axis(boxes, idx, axis=1)
    img_h = target_sizes[:, 0]
    img_w = target_sizes[:, 1]
    scale_fct = jnp.stack([img_w, img_h, img_w, img_h], axis=1)
    boxes = boxes * scale_fct[:, None, :]
    return scores, labels, boxes


# speedup vs baseline: 4.4936x; 4.4936x over previous
import functools

import jax
import jax.numpy as jnp
from jax import lax
from jax.experimental import pallas as pl
from jax.experimental.pallas import tpu as pltpu
from jax.experimental.pallas import tpu_sc as plsc

B, Q, T, C, K = 128, 900, 256, 91, 300
CPAD = 96
NB = Q * CPAD
CAP = 512

L1_BITS, L2_BITS, L3_BITS = 11, 10, 10
HIST_WORDS = 16 * (1 << L1_BITS)
NEG1_BITS = -1082130432


def _prob_body(logits_ref, pm_ref, prob_ref):
    x = logits_ref[0]
    sig = 1.0 / (1.0 + jnp.exp(-x))
    pm = pm_ref[...]
    prob = jax.lax.dot_general(
        sig, pm, (((1,), (1,)), ((), ())),
        preferred_element_type=jnp.float32)
    col = lax.broadcasted_iota(jnp.int32, (Q, CPAD), 1)
    bits = lax.bitcast_convert_type(prob, jnp.int32)
    prob_ref[0] = jnp.where(col < C, bits, NEG1_BITS)


def _compute_prob(pred_logits, positive_map):
    pm96 = jnp.concatenate(
        [positive_map, jnp.zeros((CPAD - C, T), jnp.float32)], axis=0)
    return pl.pallas_call(
        _prob_body,
        grid=(B,),
        in_specs=[
            pl.BlockSpec((1, Q, T), lambda b: (b, 0, 0)),
            pl.BlockSpec((CPAD, T), lambda b: (0, 0)),
        ],
        out_specs=pl.BlockSpec((1, Q, CPAD), lambda b: (b, 0, 0)),
        out_shape=jax.ShapeDtypeStruct((B, Q, CPAD), jnp.int32),
    )(pred_logits, pm96)


def _selector_body(nb, k, cap, rows_per_w, nc, prob_hbm, vals_hbm, idx_hbm,
                   row_v, hist_v, ov_v, oi_v):
    nv = nb // 16
    lane = lax.broadcasted_iota(jnp.int32, (16,), 0)
    ones = jnp.ones((16,), jnp.int32)
    zeros = jnp.zeros((16,), jnp.int32)
    neg1 = jnp.full((16,), NEG1_BITS, jnp.int32)
    wid = lax.axis_index("s") * nc + lax.axis_index("c")

    def clear_hist(nbins):
        def st(j, _):
            hist_v[pl.ds(j * 16, 16)] = zeros
            return _
        lax.fori_loop(0, nbins, st, 0)

    def hist_pass(shift, bits, mask_fn):
        nbins = 1 << bits
        clear_hist(nbins)

        def body_i(i, _):
            kk = row_v[pl.ds(i * 16, 16)]
            m = mask_fn(kk)
            d = lax.shift_right_logical(kk, shift) & (nbins - 1)
            d = jnp.where(m, d, 0)
            plsc.addupdate_scatter(hist_v, [lane * nbins + d], ones, mask=m)
            return _
        lax.fori_loop(0, nv, body_i, 0)

    def hist_scan(bits, kt):
        nbins = 1 << bits
        nchunk = nbins // 16

        def chunk(jj, carry):
            cum_above, b_sel, n_gt = carry
            j = nchunk - 1 - jj
            acc = zeros
            for l in range(16):
                acc = acc + hist_v[pl.ds(l * nbins + j * 16, 16)]
            tot = jnp.sum(acc)
            pre = plsc.cumsum(acc)
            suf = cum_above + tot - pre + acc
            ge = suf >= kt
            lane_sel = jnp.max(jnp.where(ge, lane, -1))
            b_here = j * 16 + lane_sel
            b_sel = jnp.where(b_sel >= 0, b_sel,
                              jnp.where(lane_sel >= 0, b_here, jnp.int32(-1)))
            n_gt = jnp.maximum(n_gt, jnp.max(jnp.where(ge, 0, suf)))
            return cum_above + tot, b_sel, n_gt

        _, b_sel, n_gt = lax.fori_loop(
            0, nchunk, chunk, (jnp.int32(0), jnp.int32(-1), jnp.int32(0)))
        return b_sel, n_gt

    def select_row(row, _):
        pltpu.sync_copy(prob_hbm.at[row], row_v)

        hist_pass(L2_BITS + L3_BITS, L1_BITS, lambda kk: kk >= 0)
        b1, ngt1 = hist_scan(L1_BITS, jnp.int32(k))
        kt2 = jnp.int32(k) - ngt1
        hist_pass(L3_BITS, L2_BITS,
                  lambda kk: (kk >= 0) &
                  (lax.shift_right_logical(kk, L2_BITS + L3_BITS) == b1))
        b2, ngt2 = hist_scan(L2_BITS, kt2)
        p21 = (b1 << L2_BITS) | b2
        kt3 = kt2 - ngt2
        hist_pass(0, L3_BITS,
                  lambda kk: (kk >= 0) &
                  (lax.shift_right_logical(kk, L3_BITS) == p21))
        b3, ngt3 = hist_scan(L3_BITS, kt3)
        vk = (p21 << L3_BITS) | b3
        eq_budget = jnp.int32(k) - (ngt1 + ngt2 + ngt3)

        def initf(j, _):
            ov_v[pl.ds(j * 16, 16)] = neg1
            oi_v[pl.ds(j * 16, 16)] = zeros
            return _
        lax.fori_loop(0, (cap + 16) // 16, initf, 0)

        def ext(i, carry):
            ptr, eq_taken = carry
            kk = row_v[pl.ds(i * 16, 16)]
            gt = kk > vk
            eq = kk == vk
            eqc = plsc.cumsum(eq.astype(jnp.int32))
            acc_eq = eq & ((eqc + eq_taken) <= eq_budget)
            accept = gt | acc_eq
            plsc.store_compressed(ov_v.at[pl.ds(ptr, 16)], kk, mask=accept)
            plsc.store_compressed(oi_v.at[pl.ds(ptr, 16)], i * 16 + lane,
                                  mask=accept)
            na = jnp.sum(accept.astype(jnp.int32))
            ne = jnp.sum(acc_eq.astype(jnp.int32))
            return ptr + na, eq_taken + ne
        lax.fori_loop(0, nv, ext, (jnp.int32(0), jnp.int32(0)))

        pltpu.sync_copy(ov_v.at[pl.ds(0, cap)], vals_hbm.at[row])
        pltpu.sync_copy(oi_v.at[pl.ds(0, cap)], idx_hbm.at[row])
        return _

    lax.fori_loop(wid * rows_per_w, (wid + 1) * rows_per_w, select_row, 0)


def _make_selector(b, nb, k, cap, nc, ns, interpret=False):
    rows_per_w = b // (nc * ns)
    mesh = plsc.VectorSubcoreMesh(
        core_axis_name="c", subcore_axis_name="s",
        num_cores=nc, num_subcores=ns)
    return pl.kernel(
        functools.partial(_selector_body, nb, k, cap, rows_per_w, nc),
        out_type=(jax.ShapeDtypeStruct((b, cap), jnp.int32),
                  jax.ShapeDtypeStruct((b, cap), jnp.int32)),
        mesh=mesh,
        scratch_types=[
            pltpu.VMEM((nb,), jnp.int32),
            pltpu.VMEM((HIST_WORDS,), jnp.int32),
            pltpu.VMEM((cap + 16,), jnp.int32),
            pltpu.VMEM((cap + 16,), jnp.int32),
        ],
        compiler_params=pltpu.CompilerParams(needs_layout_passes=False),
        interpret=interpret,
    )


def kernel(pred_logits, pred_boxes, target_sizes, positive_map):
    prob = _compute_prob(pred_logits, positive_map)
    sel = _make_selector(B, NB, K, CAP, 2, 16)
    vbits, idxs = sel(prob.reshape(B, NB))
    vals = lax.bitcast_convert_type(vbits, jnp.float32)

    scores, pos = jax.lax.top_k(vals, K)
    sidx = jnp.take_along_axis(idxs, pos, axis=1)
    topk_boxes = sidx // CPAD
    labels = sidx % CPAD

    cx, cy, w, h = (pred_boxes[..., i] for i in range(4))
    boxes = jnp.stack([cx - 0.5 * w, cy - 0.5 * h, cx + 0.5 * w, cy + 0.5 * h],
                      axis=-1)
    idx4 = jnp.repeat(topk_boxes[:, :, None], 4, axis=2)
    boxes = jnp.take_along_axis(boxes, idx4, axis=1)
    img_h = target_sizes[:, 0]
    img_w = target_sizes[:, 1]
    scale_fct = jnp.stack([img_w, img_h, img_w, img_h], axis=1)
    boxes = boxes * scale_fct[:, None, :]
    return scores, labels, boxes


# speedup vs baseline: 7.1804x; 1.5979x over previous
import functools

import jax
import jax.numpy as jnp
from jax import lax
from jax.experimental import pallas as pl
from jax.experimental.pallas import tpu as pltpu
from jax.experimental.pallas import tpu_sc as plsc

B, Q, T, C, K = 128, 900, 256, 91, 300
CPAD = 96
NB = Q * CPAD
CAP = 512
CCAP = 4096

L1_BITS, L2_BITS, L3_BITS = 11, 10, 10
HIST_WORDS = 16 * (1 << L1_BITS)
NEG1_BITS = -1082130432


def _prob_body(logits_ref, pm_ref, prob_ref):
    x = logits_ref[0]
    sig = 1.0 / (1.0 + jnp.exp(-x))
    pm = pm_ref[...]
    prob = jax.lax.dot_general(
        sig, pm, (((1,), (1,)), ((), ())),
        preferred_element_type=jnp.float32)
    col = lax.broadcasted_iota(jnp.int32, (Q, CPAD), 1)
    bits = lax.bitcast_convert_type(prob, jnp.int32)
    prob_ref[0] = jnp.where(col < C, bits, NEG1_BITS)


def _compute_prob(pred_logits, positive_map):
    pm96 = jnp.concatenate(
        [positive_map, jnp.zeros((CPAD - C, T), jnp.float32)], axis=0)
    return pl.pallas_call(
        _prob_body,
        grid=(B,),
        in_specs=[
            pl.BlockSpec((1, Q, T), lambda b: (b, 0, 0)),
            pl.BlockSpec((CPAD, T), lambda b: (0, 0)),
        ],
        out_specs=pl.BlockSpec((1, Q, CPAD), lambda b: (b, 0, 0)),
        out_shape=jax.ShapeDtypeStruct((B, Q, CPAD), jnp.int32),
    )(pred_logits, pm96)


def _selector_body(nb, k, cap, ccap, rows_per_w, nc, prob_hbm, vals_hbm,
                   idx_hbm, row_v, hist_v, cand_k, cand_i, ov_v, oi_v):
    nv = nb // 16
    UN = 4
    nbins = 1 << L1_BITS
    lane = lax.broadcasted_iota(jnp.int32, (16,), 0)
    ones = jnp.ones((16,), jnp.int32)
    zeros = jnp.zeros((16,), jnp.int32)
    neg1 = jnp.full((16,), NEG1_BITS, jnp.int32)
    true16 = jnp.ones((16,), jnp.bool_)
    wid = lax.axis_index("s") * nc + lax.axis_index("c")

    def select_row(row, _):
        pltpu.sync_copy(prob_hbm.at[row], row_v)

        def clear(j, _):
            for u in range(UN):
                hist_v[pl.ds((j * UN + u) * 16, 16)] = zeros
            return _
        lax.fori_loop(0, nbins // UN, clear, 0)

        def h1(t, _):
            for u in range(UN):
                i = t * UN + u
                kk = row_v[pl.ds(i * 16, 16)]
                m = kk >= 0
                d = jnp.where(m, lax.shift_right_logical(kk, 20), 0)
                plsc.addupdate_scatter(hist_v, [lane * nbins + d], ones,
                                       mask=m)
            return _
        lax.fori_loop(0, nv // UN, h1, 0)

        def chunk(jj, carry):
            cum_above, b_sel, n_ge = carry
            j = nbins // 16 - 1 - jj
            acc = zeros
            for l in range(16):
                acc = acc + hist_v[pl.ds(l * nbins + j * 16, 16)]
            tot = jnp.sum(acc)
            pre = plsc.cumsum(acc)
            suf = cum_above + tot - pre + acc
            ge = suf >= k
            lane_sel = jnp.max(jnp.where(ge, lane, -1))
            b_here = j * 16 + lane_sel
            b_sel = jnp.where(b_sel >= 0, b_sel,
                              jnp.where(lane_sel >= 0, b_here, jnp.int32(-1)))
            n_ge = jnp.minimum(n_ge, jnp.min(jnp.where(ge, suf, 1 << 30)))
            return cum_above + tot, b_sel, n_ge

        _, b1, n_ge1 = lax.fori_loop(
            0, nbins // 16, chunk,
            (jnp.int32(0), jnp.int32(-1), jnp.int32(1 << 30)))
        base = b1 << 20

        def g(t, ptrv):
            for u in range(UN):
                i = t * UN + u
                kk = row_v[pl.ds(i * 16, 16)]
                m = kk >= base
                pos = ptrv + plsc.cumsum(m.astype(jnp.int32)) - 1
                pos = jnp.minimum(pos, ccap + 16)
                plsc.store_scatter(cand_k, [pos], kk, mask=m)
                plsc.store_scatter(cand_i, [pos], i * 16 + lane, mask=m)
                ptrv = ptrv + plsc.all_reduce_population_count(m)
            return ptrv
        lax.fori_loop(0, nv // UN, g, zeros)

        def initf(j, _):
            ov_v[pl.ds(j * 16, 16)] = neg1
            oi_v[pl.ds(j * 16, 16)] = zeros
            return _
        lax.fori_loop(0, (cap + 16) // 16, initf, 0)

        def finish(src_k, load_idx, ntrips):
            def bs(j, cur):
                t = cur | lax.shift_left(jnp.int32(1), 19 - j)

                def sweep(i, acc):
                    kk = src_k[pl.ds(i * 16, 16)]
                    return acc + (kk >= t).astype(jnp.int32)
                cnt = jnp.sum(lax.fori_loop(0, ntrips, sweep, zeros))
                return jnp.where(cnt >= k, t, cur)
            vk = lax.fori_loop(0, 20, bs, base)

            def sweep2(i, acc):
                kk = src_k[pl.ds(i * 16, 16)]
                return acc + (kk > vk).astype(jnp.int32)
            n_gt = jnp.sum(lax.fori_loop(0, ntrips, sweep2, zeros))
            eqb = k - n_gt

            def ext(i, carry):
                ptrv, eqt = carry
                kk = src_k[pl.ds(i * 16, 16)]
                iv = load_idx(i)
                gt = kk > vk
                eq = kk == vk
                eqc = plsc.cumsum(eq.astype(jnp.int32))
                acc_eq = eq & ((eqc + eqt) <= eqb)
                accept = gt | acc_eq
                pos = ptrv + plsc.cumsum(accept.astype(jnp.int32)) - 1
                plsc.store_scatter(ov_v, [pos], kk, mask=accept)
                plsc.store_scatter(oi_v, [pos], iv, mask=accept)
                ptrv = ptrv + plsc.all_reduce_population_count(accept)
                eqt = eqt + plsc.all_reduce_population_count(acc_eq)
                return ptrv, eqt
            lax.fori_loop(0, ntrips, ext, (zeros, zeros))
            return jnp.int32(0)

        def compact_branch():
            plsc.store_scatter(cand_k, [n_ge1 + lane], neg1, mask=true16)
            return finish(cand_k, lambda i: cand_i[pl.ds(i * 16, 16)],
                          (n_ge1 + 15) // 16)

        def full_branch():
            return finish(row_v, lambda i: i * 16 + lane, nv)

        lax.cond(n_ge1 <= ccap, compact_branch, full_branch)

        pltpu.sync_copy(ov_v.at[pl.ds(0, cap)], vals_hbm.at[row])
        pltpu.sync_copy(oi_v.at[pl.ds(0, cap)], idx_hbm.at[row])
        return _

    lax.fori_loop(wid * rows_per_w, (wid + 1) * rows_per_w, select_row, 0)


def _make_selector(b, nb, k, cap, ccap, nc, ns, interpret=False):
    rows_per_w = b // (nc * ns)
    mesh = plsc.VectorSubcoreMesh(
        core_axis_name="c", subcore_axis_name="s",
        num_cores=nc, num_subcores=ns)
    return pl.kernel(
        functools.partial(_selector_body, nb, k, cap, ccap, rows_per_w, nc),
        out_type=(jax.ShapeDtypeStruct((b, cap), jnp.int32),
                  jax.ShapeDtypeStruct((b, cap), jnp.int32)),
        mesh=mesh,
        scratch_types=[
            pltpu.VMEM((nb,), jnp.int32),
            pltpu.VMEM((HIST_WORDS,), jnp.int32),
            pltpu.VMEM((ccap + 32,), jnp.int32),
            pltpu.VMEM((ccap + 32,), jnp.int32),
            pltpu.VMEM((cap + 16,), jnp.int32),
            pltpu.VMEM((cap + 16,), jnp.int32),
        ],
        compiler_params=pltpu.CompilerParams(needs_layout_passes=False),
        interpret=interpret,
    )


def kernel(pred_logits, pred_boxes, target_sizes, positive_map):
    prob = _compute_prob(pred_logits, positive_map)
    sel = _make_selector(B, NB, K, CAP, CCAP, 2, 16)
    vbits, idxs = sel(prob.reshape(B, NB))
    vals = lax.bitcast_convert_type(vbits, jnp.float32)

    scores, pos = jax.lax.top_k(vals, K)
    sidx = jnp.take_along_axis(idxs, pos, axis=1)
    topk_boxes = sidx // CPAD
    labels = sidx % CPAD

    cx, cy, w, h = (pred_boxes[..., i] for i in range(4))
    boxes = jnp.stack([cx - 0.5 * w, cy - 0.5 * h, cx + 0.5 * w, cy + 0.5 * h],
                      axis=-1)
    idx4 = jnp.repeat(topk_boxes[:, :, None], 4, axis=2)
    boxes = jnp.take_along_axis(boxes, idx4, axis=1)
    img_h = target_sizes[:, 0]
    img_w = target_sizes[:, 1]
    scale_fct = jnp.stack([img_w, img_h, img_w, img_h], axis=1)
    boxes = boxes * scale_fct[:, None, :]
    return scores, labels, boxes


# speedup vs baseline: 11.2071x; 1.5608x over previous
import functools

import jax
import jax.numpy as jnp
from jax import lax
from jax.experimental import pallas as pl
from jax.experimental.pallas import tpu as pltpu
from jax.experimental.pallas import tpu_sc as plsc

B, Q, T, C, K = 128, 900, 256, 91, 300
CPAD = 96
NB = Q * CPAD
QPAD = 1024
CAP = 512
CCAP = 8192
NEG1_BITS = -1082130432


def _prob_body(logits_ref, pm_ref, prob_ref, qmax_ref):
    x = logits_ref[0]
    sig = 1.0 / (1.0 + jnp.exp(-x))
    pm = pm_ref[...]
    prob = jax.lax.dot_general(
        sig, pm, (((1,), (1,)), ((), ())),
        preferred_element_type=jnp.float32)
    col = lax.broadcasted_iota(jnp.int32, (Q, CPAD), 1)
    bits = lax.bitcast_convert_type(prob, jnp.int32)
    bits = jnp.where(col < C, bits, NEG1_BITS)
    prob_ref[0] = bits
    qmax = jnp.max(bits, axis=1)
    qmax_ref[0, 0] = jnp.concatenate(
        [qmax, jnp.full((QPAD - Q,), NEG1_BITS, jnp.int32)])


def _compute_prob(pred_logits, positive_map):
    pm96 = jnp.concatenate(
        [positive_map, jnp.zeros((CPAD - C, T), jnp.float32)], axis=0)
    return pl.pallas_call(
        _prob_body,
        grid=(B,),
        in_specs=[
            pl.BlockSpec((1, Q, T), lambda b: (b, 0, 0)),
            pl.BlockSpec((CPAD, T), lambda b: (0, 0)),
        ],
        out_specs=[pl.BlockSpec((1, Q, CPAD), lambda b: (b, 0, 0)),
                   pl.BlockSpec((1, 1, QPAD), lambda b: (b, 0, 0))],
        out_shape=[jax.ShapeDtypeStruct((B, Q, CPAD), jnp.int32),
                   jax.ShapeDtypeStruct((B, 1, QPAD), jnp.int32)],
    )(pred_logits, pm96)


def _selector_body(nb, k, cap, ccap, qpad, rows_per_w, nc, prob_hbm,
                   qmax_hbm, vals_hbm, idx_hbm, row_v, qm_v, cand_k, cand_i,
                   ov_v, oi_v):
    nv = nb // 16
    UN = 8
    lane = lax.broadcasted_iota(jnp.int32, (16,), 0)
    zeros = jnp.zeros((16,), jnp.int32)
    neg1 = jnp.full((16,), NEG1_BITS, jnp.int32)
    true16 = jnp.ones((16,), jnp.bool_)
    wid = lax.axis_index("s") * nc + lax.axis_index("c")

    def select_row(row, _):
        pltpu.sync_copy(prob_hbm.at[row], row_v)
        pltpu.sync_copy(qmax_hbm.at[row], qm_v)

        def bs_m(j, cur):
            t = cur | lax.shift_left(jnp.int32(1), 30 - j)

            def sweep(i, acc):
                return acc + (qm_v[pl.ds(i * 16, 16)] >= t).astype(jnp.int32)
            cnt = jnp.sum(lax.fori_loop(0, qpad // 16, sweep, zeros))
            return jnp.where(cnt >= k, t, cur)
        m = lax.fori_loop(0, 31, bs_m, jnp.int32(0))

        def g(t, ptrv):
            kks = []
            ms = []
            for u in range(UN):
                i = t * UN + u
                kk = row_v[pl.ds(i * 16, 16)]
                kks.append(kk)
                ms.append(kk >= m)
            anym = ms[0]
            for u in range(1, UN):
                anym = anym | ms[u]

            def do_store():
                pv = ptrv
                for u in range(UN):
                    pos = pv + plsc.cumsum(ms[u].astype(jnp.int32)) - 1
                    pos = jnp.minimum(pos, ccap + 16)
                    plsc.store_scatter(cand_k, [pos], kks[u], mask=ms[u])
                    plsc.store_scatter(cand_i, [pos], (t * UN + u) * 16 + lane,
                                       mask=ms[u])
                    pv = pv + plsc.all_reduce_population_count(ms[u])
                return pv
            return lax.cond(jnp.any(anym), do_store, lambda: ptrv)
        ptrv = lax.fori_loop(0, nv // UN, g, zeros)
        n_cand = jnp.max(ptrv)

        def initf(j, _):
            ov_v[pl.ds(j * 16, 16)] = neg1
            oi_v[pl.ds(j * 16, 16)] = zeros
            return _
        lax.fori_loop(0, (cap + 16) // 16, initf, 0)

        def finish(src_k, load_idx, ntrips):
            def bs(j, cur):
                t = cur | lax.shift_left(jnp.int32(1), 30 - j)

                def sweep(i, acc):
                    kk = src_k[pl.ds(i * 16, 16)]
                    return acc + (kk >= t).astype(jnp.int32)
                cnt = jnp.sum(lax.fori_loop(0, ntrips, sweep, zeros))
                return jnp.where(cnt >= k, t, cur)
            vk = lax.fori_loop(0, 31, bs, jnp.int32(0))

            def sweep2(i, acc):
                kk = src_k[pl.ds(i * 16, 16)]
                return acc + (kk > vk).astype(jnp.int32)
            n_gt = jnp.sum(lax.fori_loop(0, ntrips, sweep2, zeros))
            eqb = k - n_gt

            def ext(i, carry):
                ptrv, eqt = carry
                kk = src_k[pl.ds(i * 16, 16)]
                iv = load_idx(i)
                gt = kk > vk
                eq = kk == vk
                eqc = plsc.cumsum(eq.astype(jnp.int32))
                acc_eq = eq & ((eqc + eqt) <= eqb)
                accept = gt | acc_eq
                pos = ptrv + plsc.cumsum(accept.astype(jnp.int32)) - 1
                plsc.store_scatter(ov_v, [pos], kk, mask=accept)
                plsc.store_scatter(oi_v, [pos], iv, mask=accept)
                ptrv = ptrv + plsc.all_reduce_population_count(accept)
                eqt = eqt + plsc.all_reduce_population_count(acc_eq)
                return ptrv, eqt
            lax.fori_loop(0, ntrips, ext, (zeros, zeros))
            return jnp.int32(0)

        def compact_branch():
            plsc.store_scatter(cand_k, [n_cand + lane], neg1, mask=true16)
            return finish(cand_k, lambda i: cand_i[pl.ds(i * 16, 16)],
                          (n_cand + 15) // 16)

        def full_branch():
            return finish(row_v, lambda i: i * 16 + lane, nv)

        lax.cond(n_cand <= ccap, compact_branch, full_branch)

        pltpu.sync_copy(ov_v.at[pl.ds(0, cap)], vals_hbm.at[row])
        pltpu.sync_copy(oi_v.at[pl.ds(0, cap)], idx_hbm.at[row])
        return _

    lax.fori_loop(wid * rows_per_w, (wid + 1) * rows_per_w, select_row, 0)


def _make_selector(b, nb, k, cap, ccap, qpad, nc, ns, interpret=False):
    rows_per_w = b // (nc * ns)
    mesh = plsc.VectorSubcoreMesh(
        core_axis_name="c", subcore_axis_name="s",
        num_cores=nc, num_subcores=ns)
    return pl.kernel(
        functools.partial(_selector_body, nb, k, cap, ccap, qpad,
                          rows_per_w, nc),
        out_type=(jax.ShapeDtypeStruct((b, cap), jnp.int32),
                  jax.ShapeDtypeStruct((b, cap), jnp.int32)),
        mesh=mesh,
        scratch_types=[
            pltpu.VMEM((nb,), jnp.int32),
            pltpu.VMEM((qpad,), jnp.int32),
            pltpu.VMEM((ccap + 32,), jnp.int32),
            pltpu.VMEM((ccap + 32,), jnp.int32),
            pltpu.VMEM((cap + 16,), jnp.int32),
            pltpu.VMEM((cap + 16,), jnp.int32),
        ],
        compiler_params=pltpu.CompilerParams(needs_layout_passes=False),
        interpret=interpret,
    )


def kernel(pred_logits, pred_boxes, target_sizes, positive_map):
    prob, qmax = _compute_prob(pred_logits, positive_map)
    sel = _make_selector(B, NB, K, CAP, CCAP, QPAD, 2, 16)
    vbits, idxs = sel(prob.reshape(B, NB), qmax.reshape(B, QPAD))
    vals = lax.bitcast_convert_type(vbits, jnp.float32)

    scores, pos = jax.lax.top_k(vals, K)
    sidx = jnp.take_along_axis(idxs, pos, axis=1)
    topk_boxes = sidx // CPAD
    labels = sidx % CPAD

    cx, cy, w, h = (pred_boxes[..., i] for i in range(4))
    boxes = jnp.stack([cx - 0.5 * w, cy - 0.5 * h, cx + 0.5 * w, cy + 0.5 * h],
                      axis=-1)
    idx4 = jnp.repeat(topk_boxes[:, :, None], 4, axis=2)
    boxes = jnp.take_along_axis(boxes, idx4, axis=1)
    img_h = target_sizes[:, 0]
    img_w = target_sizes[:, 1]
    scale_fct = jnp.stack([img_w, img_h, img_w, img_h], axis=1)
    boxes = boxes * scale_fct[:, None, :]
    return scores, labels, boxes
